# baseline (speedup 1.0000x reference)
import functools
import math

import jax
import jax.numpy as jnp
from jax import lax
from jax.experimental import pallas as pl
from jax.experimental.pallas import tpu as pltpu
from jax.experimental.pallas import tpu_sc as plsc

VOCAB = 100000
DIM = 64
D2 = DIM * 2
D4 = DIM * 4
MAX_SEQ = 512
B = 1024
S = 200
N = B * S
NW = 32
CH = 100
G = S
PER_W = N // NW
NG = PER_W // G
NCH = PER_W // CH
IBLK = 16
NBLK = NCH // IBLK
GPB = IBLK // 2
LANES = 16
NBUF = 2
RU = 2


def _pos_table():
    position = jnp.arange(MAX_SEQ, dtype=jnp.float32)[:, None]
    div_term = jnp.exp(
        jnp.arange(0, DIM, 2).astype(jnp.float32) * (-math.log(10000.0) / DIM)
    )
    pe_sin = jnp.sin(position * div_term)
    pe_cos = jnp.cos(position * div_term)
    pe_real = jnp.zeros((MAX_SEQ, DIM), jnp.float32)
    pe_real = pe_real.at[:, 0::2].set(pe_sin)
    pe_real = pe_real.at[:, 1::2].set(pe_cos)
    pe_imag = jnp.zeros((MAX_SEQ, DIM), jnp.float32)
    pe_imag = pe_imag.at[:, 0::2].set(pe_cos)
    pe_imag = pe_imag.at[:, 1::2].set(-pe_sin)
    return jnp.concatenate([pe_real, pe_imag], axis=-1)[:S]


def _sc_embed(tok2d, omega_table, pi_table, pos):
    mesh = plsc.VectorSubcoreMesh(core_axis_name="c", subcore_axis_name="s")

    @functools.partial(
        pl.kernel,
        out_type=jax.ShapeDtypeStruct((N, D4), jnp.float32),
        mesh=mesh,
        scratch_types=[
            pltpu.VMEM((IBLK, CH), jnp.int32),
            pltpu.VMEM((S, D2), jnp.float32),
            [pltpu.VMEM((G, D2), jnp.float32)] * NBUF,
            [pltpu.VMEM((G, D2), jnp.float32)] * NBUF,
            [pltpu.SemaphoreType.DMA] * NBUF,
            [pltpu.SemaphoreType.DMA] * NBUF,
            pltpu.SemaphoreType.DMA,
        ],
    )
    def k(tok_hbm, omega_hbm, pi_hbm, pos_hbm, out_hbm,
          idx_v, pos_v, om_v, pi_v, sem_o, sem_p, sem_w):
        wid = lax.axis_index("s") * 2 + lax.axis_index("c")
        base = wid * PER_W
        pltpu.sync_copy(pos_hbm, pos_v)

        def load_idx(blk):
            pltpu.sync_copy(tok_hbm.at[pl.ds(wid * NCH + blk * IBLK, IBLK)],
                            idx_v)

        def gathers(c, b):
            pltpu.async_copy(omega_hbm.at[idx_v.at[c]],
                             om_v[b].at[pl.ds(0, CH)], sem_o[b])
            pltpu.async_copy(omega_hbm.at[idx_v.at[c + 1]],
                             om_v[b].at[pl.ds(CH, CH)], sem_o[b])
            pltpu.async_copy(pi_hbm.at[idx_v.at[c]],
                             pi_v[b].at[pl.ds(0, CH)], sem_p[b])
            pltpu.async_copy(pi_hbm.at[idx_v.at[c + 1]],
                             pi_v[b].at[pl.ds(CH, CH)], sem_p[b])

        def wait_gathers(b):
            pltpu.make_async_copy(
                omega_hbm.at[pl.ds(0, G)], om_v[b], sem_o[b]).wait()
            pltpu.make_async_copy(
                pi_hbm.at[pl.ds(0, G)], pi_v[b], sem_p[b]).wait()

        load_idx(0)
        gathers(0, 0)

        def block_body(blk, carry):
            for gb in range(GPB):
                g = blk * GPB + gb
                b = gb % NBUF
                nb = (gb + 1) % NBUF
                if gb == GPB - 1:
                    wait_gathers(b)

                    @pl.when(blk < NBLK - 1)
                    def _():
                        load_idx(blk + 1)
                        gathers(0, nb)
                else:
                    gathers(2 * (gb + 1), nb)
                    wait_gathers(b)

                def row_body(jj, carry2):
                    for r in range(RU):
                        j = jj * RU + r
                        for h in range(D2 // LANES):
                            sl = pl.ds(h * LANES, LANES)
                            p = pos_v[j, sl]
                            om_v[b][j, sl] = om_v[b][j, sl] + p
                            pi_v[b][j, sl] = pi_v[b][j, sl] + p
                    return carry2

                lax.fori_loop(0, G // RU, row_body, 0)
                r0 = base + g * G
                cp_a = pltpu.async_copy(
                    om_v[b], out_hbm.at[pl.ds(r0, G), pl.ds(0, D2)], sem_w)
                cp_b = pltpu.async_copy(
                    pi_v[b], out_hbm.at[pl.ds(r0, G), pl.ds(D2, D2)], sem_w)
                cp_a.wait()
                cp_b.wait()
            return carry

        lax.fori_loop(0, NBLK, block_body, 0)

    return k(tok2d, omega_table, pi_table, pos)


def kernel(token_ids, omega_table, pi_table):
    tok2d = token_ids.reshape(N // CH, CH).astype(jnp.int32)
    pos = _pos_table()
    out = _sc_embed(tok2d, omega_table, pi_table, pos)
    return out.reshape(B, S, D4)
